# Initial kernel scaffold; baseline (speedup 1.0000x reference)
import jax
import jax.numpy as jnp
from jax.experimental import pallas as pl


def kernel(input_ids, weight):
    raise NotImplementedError("write your pallas kernel here")


# speedup vs baseline: 1.5093x; 1.5093x over previous
import functools

import jax
import jax.numpy as jnp
from jax import lax
from jax.experimental import pallas as pl
from jax.experimental.pallas import tpu as pltpu
from jax.experimental.pallas import tpu_sc as plsc

B_TOTAL = 4096 * 50
D = 768
NUM_WORKERS = 32
B_PER_W = B_TOTAL // NUM_WORKERS
CHUNK = 64
N_CHUNKS = B_PER_W // CHUNK

_mesh = plsc.VectorSubcoreMesh(core_axis_name="c", subcore_axis_name="s")


@functools.partial(
    pl.kernel,
    mesh=_mesh,
    out_type=jax.ShapeDtypeStruct((B_TOTAL, D), jnp.float32),
    scratch_types=[
        pltpu.VMEM((B_PER_W,), jnp.int32),
        pltpu.VMEM((CHUNK, D), jnp.float32),
        pltpu.SemaphoreType.DMA,
    ],
)
def _gather_sc(ids_hbm, table_hbm, out_hbm, idx_v, buf, gsem):
    cid = lax.axis_index("c")
    sid = lax.axis_index("s")
    wid = sid * 2 + cid
    base = wid * B_PER_W

    pltpu.sync_copy(ids_hbm.at[pl.ds(base, B_PER_W)], idx_v)

    def step(i, carry):
        off = i * CHUNK
        pltpu.async_copy(
            table_hbm.at[idx_v.at[pl.ds(off, CHUNK)]], buf, gsem
        ).wait()
        pltpu.sync_copy(buf, out_hbm.at[pl.ds(base + off, CHUNK)])
        return carry

    lax.fori_loop(0, N_CHUNKS, step, 0)


def kernel(input_ids, weight):
    ids_flat = input_ids.reshape(-1).astype(jnp.int32)
    out = _gather_sc(ids_flat, weight)
    return out.reshape(input_ids.shape + (D,))


# speedup vs baseline: 1.5566x; 1.0313x over previous
import functools

import jax
import jax.numpy as jnp
from jax import lax
from jax.experimental import pallas as pl
from jax.experimental.pallas import tpu as pltpu
from jax.experimental.pallas import tpu_sc as plsc

B_TOTAL = 4096 * 50
D = 768
NUM_WORKERS = 32
B_PER_W = B_TOTAL // NUM_WORKERS
CHUNK = 64
NBUF = 2
N_CHUNKS = B_PER_W // CHUNK
N_GROUPS = N_CHUNKS // NBUF

_mesh = plsc.VectorSubcoreMesh(core_axis_name="c", subcore_axis_name="s")


@functools.partial(
    pl.kernel,
    mesh=_mesh,
    out_type=jax.ShapeDtypeStruct((B_TOTAL, D), jnp.float32),
    scratch_types=[
        pltpu.VMEM((B_PER_W,), jnp.int32),
        pltpu.VMEM((CHUNK, D), jnp.float32),
        pltpu.VMEM((CHUNK, D), jnp.float32),
        pltpu.SemaphoreType.DMA,
        pltpu.SemaphoreType.DMA,
        pltpu.SemaphoreType.DMA,
        pltpu.SemaphoreType.DMA,
    ],
)
def _gather_sc(ids_hbm, table_hbm, out_hbm, idx_v, buf0, buf1, g0, g1, s0, s1):
    cid = lax.axis_index("c")
    sid = lax.axis_index("s")
    wid = sid * 2 + cid
    base = wid * B_PER_W

    bufs = (buf0, buf1)
    gsems = (g0, g1)
    ssems = (s0, s1)

    pltpu.sync_copy(ids_hbm.at[pl.ds(base, B_PER_W)], idx_v)

    def gather_copy(chunk_id, b):
        off = chunk_id * CHUNK
        return pltpu.make_async_copy(
            table_hbm.at[idx_v.at[pl.ds(off, CHUNK)]], bufs[b], gsems[b]
        )

    def scatter_copy(chunk_id, b):
        off = chunk_id * CHUNK
        return pltpu.make_async_copy(
            bufs[b], out_hbm.at[pl.ds(base + off, CHUNK)], ssems[b]
        )

    for b in range(NBUF):
        gather_copy(b, b).start()

    def group(j, carry):
        c0 = j * NBUF
        for b in range(NBUF):
            gather_copy(c0 + b, b).wait()
            scatter_copy(c0 + b, b).start()
        for b in range(NBUF):
            scatter_copy(c0 + b, b).wait()
            gather_copy(c0 + NBUF + b, b).start()
        return carry

    lax.fori_loop(0, N_GROUPS - 1, group, 0)

    c0 = (N_GROUPS - 1) * NBUF
    for b in range(NBUF):
        gather_copy(c0 + b, b).wait()
        scatter_copy(c0 + b, b).start()
    for b in range(NBUF):
        scatter_copy(c0 + b, b).wait()


def kernel(input_ids, weight):
    ids_flat = input_ids.reshape(-1).astype(jnp.int32)
    out = _gather_sc(ids_flat, weight)
    return out.reshape(input_ids.shape + (D,))
